# baseline (speedup 1.0000x reference)
import functools

import jax
import jax.numpy as jnp
from jax import lax
from jax.experimental import pallas as pl
from jax.experimental.pallas import tpu as pltpu


def _mono_kernel(a_ref, x_ref, w1_ref, b1_ref, w2_ref, b2_ref,
                 out_ref, cache_ref, xs_ref, dbf_ref, d_ref, acc_ref, v_ref,
                 *, s, t, inv_n):
    k = pl.program_id(0)
    f32 = jnp.float32
    n = d_ref.shape[0]

    @pl.when(k == 0)
    def _():
        acc_ref[...] = jnp.zeros_like(acc_ref)
        v_ref[...] = jnp.zeros_like(v_ref)

    def emit_dot(jj, xs, dbf):
        cj = cache_ref[:, jj * t:(jj + 1) * t]
        acc_ref[jj * t:(jj + 1) * t, :] += lax.dot_general(
            cj, xs, (((0,), (0,)), ((), ())), preferred_element_type=f32)
        v_ref[:, jj * t:(jj + 1) * t] += lax.dot_general(
            dbf, cj, (((0,), (0,)), ((), ())), preferred_element_type=f32)

    def ingest(do_dots):
        rs = jnp.zeros((t, 1), f32)
        cw = min(512, t)
        cpj = t // cw
        xs_p = xs_ref[...] if do_dots else None
        dbf_p = dbf_ref[...] if do_dots else None
        for jj in range(s):
            if do_dots:
                emit_dot(jj, xs_p, dbf_p)
            for c2 in range(cpj):
                lo = jj * t + c2 * cw
                chunk = a_ref[:, lo:lo + cw]
                rs = rs + jnp.sum(chunk, axis=1, keepdims=True)
                cache_ref[:, lo:lo + cw] = chunk.astype(jnp.bfloat16)
        d = lax.rsqrt(1.0 + rs)
        d_ref[pl.ds(k * t, t), :] = d
        xs_f = d * x_ref[...]
        acc_ref[pl.ds(k * t, t), :] += xs_f
        xs_ref[...] = xs_f.astype(jnp.bfloat16)
        dbf_ref[...] = d.astype(jnp.bfloat16)

    @pl.when(k == 0)
    def _():
        ingest(do_dots=False)

    @pl.when(k > 0)
    def _():
        ingest(do_dots=True)

    @pl.when(k == s - 1)
    def _():
        xs = xs_ref[...]
        dbf = dbf_ref[...]
        for jj in range(s):
            emit_dot(jj, xs, dbf)
        p = jnp.zeros_like(out_ref)
        for i in range(s):
            d_i = d_ref[i * t:(i + 1) * t, :]
            h = jnp.dot(d_i * acc_ref[i * t:(i + 1) * t, :], w1_ref[...],
                        preferred_element_type=f32) + b1_ref[...]
            h = jnp.maximum(h, 0.0)
            m = jnp.dot(d_i * h, w2_ref[...], preferred_element_type=f32)
            p = (p + jnp.dot(v_ref[:, i * t:(i + 1) * t], m,
                             preferred_element_type=f32)
                 + lax.dot_general(d_i, m, (((0,), (0,)), ((), ())),
                                   preferred_element_type=f32))
        out_ref[...] = p * inv_n + b2_ref[...]


def _mono_pass(a, x, w1, b1, w2, b2, t):
    n, f_in = x.shape
    h_feats = w1.shape[1]
    c = w2.shape[1]
    s = n // t
    body = functools.partial(_mono_kernel, s=s, t=t, inv_n=1.0 / n)
    return pl.pallas_call(
        body,
        out_shape=jax.ShapeDtypeStruct((1, c), jnp.float32),
        grid_spec=pltpu.PrefetchScalarGridSpec(
            num_scalar_prefetch=0,
            grid=(s,),
            in_specs=[
                pl.BlockSpec((t, n), lambda k: (k, 0)),
                pl.BlockSpec((t, f_in), lambda k: (k, 0)),
                pl.BlockSpec((f_in, h_feats), lambda k: (0, 0)),
                pl.BlockSpec((1, h_feats), lambda k: (0, 0)),
                pl.BlockSpec((h_feats, c), lambda k: (0, 0)),
                pl.BlockSpec((1, c), lambda k: (0, 0)),
            ],
            out_specs=pl.BlockSpec((1, c), lambda k: (0, 0)),
            scratch_shapes=[
                pltpu.VMEM((t, n), jnp.bfloat16),
                pltpu.VMEM((t, f_in), jnp.bfloat16),
                pltpu.VMEM((t, 1), jnp.bfloat16),
                pltpu.VMEM((n, 1), jnp.float32),
                pltpu.VMEM((n, f_in), jnp.float32),
                pltpu.VMEM((1, n), jnp.float32),
            ],
        ),
        compiler_params=pltpu.CompilerParams(
            dimension_semantics=("arbitrary",)),
    )(a, x, w1, b1, w2, b2)


def kernel(adj, features, w1, b1, w2, b2):
    n = adj.shape[0]
    t = 1024 if n % 1024 == 0 else n
    return _mono_pass(adj, features, w1, b1, w2, b2, t)


# speedup vs baseline: 1.0973x; 1.0973x over previous
import functools

import jax
import jax.numpy as jnp
from jax import lax
from jax.experimental import pallas as pl
from jax.experimental.pallas import tpu as pltpu


def _mono_kernel(a_ref, x_ref, w1_ref, b1_ref, w2_ref, b2_ref,
                 out_ref, ca_ref, cb_ref, xsa_ref, xsb_ref, dba_ref, dbb_ref,
                 d_ref, acc_ref, v_ref, *, s, t, inv_n):
    k = pl.program_id(0)
    f32 = jnp.float32
    n = d_ref.shape[0]

    @pl.when(k == 0)
    def _():
        acc_ref[...] = jnp.zeros_like(acc_ref)
        v_ref[...] = jnp.zeros_like(v_ref)

    def emit_dot(cache_ref, jj, xs, dbf):
        cj = cache_ref[:, jj * t:(jj + 1) * t]
        acc_ref[jj * t:(jj + 1) * t, :] += lax.dot_general(
            cj, xs, (((0,), (0,)), ((), ())), preferred_element_type=f32)
        v_ref[:, jj * t:(jj + 1) * t] += lax.dot_general(
            dbf, cj, (((0,), (0,)), ((), ())), preferred_element_type=f32)

    def ingest(c_out, xs_out, db_out, prev):
        if prev is not None:
            c_in, xs_in, db_in = prev
            xs_p = xs_in[...]
            dbf_p = db_in[...]
        rs = jnp.zeros((t, 1), f32)
        cw = min(512, t)
        cpj = t // cw
        for jj in range(s):
            if prev is not None:
                emit_dot(c_in, jj, xs_p, dbf_p)
            for c2 in range(cpj):
                lo = jj * t + c2 * cw
                chunk = a_ref[:, lo:lo + cw]
                rs = rs + jnp.sum(chunk, axis=1, keepdims=True)
                c_out[:, lo:lo + cw] = chunk.astype(jnp.bfloat16)
        d = lax.rsqrt(1.0 + rs)
        d_ref[pl.ds(k * t, t), :] = d
        xs_f = d * x_ref[...]
        acc_ref[pl.ds(k * t, t), :] += xs_f
        xs_out[...] = xs_f.astype(jnp.bfloat16)
        db_out[...] = d.astype(jnp.bfloat16)

    even = (ca_ref, xsa_ref, dba_ref)
    odd = (cb_ref, xsb_ref, dbb_ref)
    parity = lax.rem(k, 2)

    @pl.when(k == 0)
    def _():
        ingest(*even, prev=None)

    @pl.when(jnp.logical_and(k > 0, parity == 0))
    def _():
        ingest(*even, prev=odd)

    @pl.when(parity == 1)
    def _():
        ingest(*odd, prev=even)

    @pl.when(k == s - 1)
    def _():
        c_l, xs_l, db_l = even if (s - 1) % 2 == 0 else odd
        xs = xs_l[...]
        dbf = db_l[...]
        for jj in range(s):
            emit_dot(c_l, jj, xs, dbf)
        p = jnp.zeros_like(out_ref)
        for i in range(s):
            d_i = d_ref[i * t:(i + 1) * t, :]
            h = jnp.dot(d_i * acc_ref[i * t:(i + 1) * t, :], w1_ref[...],
                        preferred_element_type=f32) + b1_ref[...]
            h = jnp.maximum(h, 0.0)
            m = jnp.dot(d_i * h, w2_ref[...], preferred_element_type=f32)
            p = (p + jnp.dot(v_ref[:, i * t:(i + 1) * t], m,
                             preferred_element_type=f32)
                 + lax.dot_general(d_i, m, (((0,), (0,)), ((), ())),
                                   preferred_element_type=f32))
        out_ref[...] = p * inv_n + b2_ref[...]


def _mono_pass(a, x, w1, b1, w2, b2, t):
    n, f_in = x.shape
    h_feats = w1.shape[1]
    c = w2.shape[1]
    s = n // t
    body = functools.partial(_mono_kernel, s=s, t=t, inv_n=1.0 / n)
    return pl.pallas_call(
        body,
        out_shape=jax.ShapeDtypeStruct((1, c), jnp.float32),
        grid_spec=pltpu.PrefetchScalarGridSpec(
            num_scalar_prefetch=0,
            grid=(s,),
            in_specs=[
                pl.BlockSpec((t, n), lambda k: (k, 0)),
                pl.BlockSpec((t, f_in), lambda k: (k, 0)),
                pl.BlockSpec((f_in, h_feats), lambda k: (0, 0)),
                pl.BlockSpec((1, h_feats), lambda k: (0, 0)),
                pl.BlockSpec((h_feats, c), lambda k: (0, 0)),
                pl.BlockSpec((1, c), lambda k: (0, 0)),
            ],
            out_specs=pl.BlockSpec((1, c), lambda k: (0, 0)),
            scratch_shapes=[
                pltpu.VMEM((t, n), jnp.bfloat16),
                pltpu.VMEM((t, n), jnp.bfloat16),
                pltpu.VMEM((t, f_in), jnp.bfloat16),
                pltpu.VMEM((t, f_in), jnp.bfloat16),
                pltpu.VMEM((t, 1), jnp.bfloat16),
                pltpu.VMEM((t, 1), jnp.bfloat16),
                pltpu.VMEM((n, 1), jnp.float32),
                pltpu.VMEM((n, f_in), jnp.float32),
                pltpu.VMEM((1, n), jnp.float32),
            ],
        ),
        compiler_params=pltpu.CompilerParams(
            dimension_semantics=("arbitrary",)),
    )(a, x, w1, b1, w2, b2)


def kernel(adj, features, w1, b1, w2, b2):
    n = adj.shape[0]
    t = 512 if n % 512 == 0 else n
    return _mono_pass(adj, features, w1, b1, w2, b2, t)
